# baseline (speedup 1.0000x reference)
import functools

import jax
import jax.numpy as jnp
from jax import lax
from jax.experimental import pallas as pl
from jax.experimental.pallas import tpu as pltpu
from jax.experimental.pallas import tpu_sc as plsc

EMBED = 64
CHUNK = 128
NUM_WORKERS = 32


@functools.lru_cache(maxsize=None)
def _make_gather(n_rows: int):
    assert n_rows % (NUM_WORKERS * CHUNK) == 0
    chunks_per_worker = n_rows // (NUM_WORKERS * CHUNK)
    rows_per_worker = chunks_per_worker * CHUNK
    mesh = plsc.VectorSubcoreMesh(core_axis_name="c", subcore_axis_name="s")

    @functools.partial(
        pl.kernel,
        mesh=mesh,
        out_type=jax.ShapeDtypeStruct((n_rows, EMBED), jnp.float32),
        scratch_types=[
            pltpu.VMEM((chunks_per_worker, CHUNK), jnp.int32),
            pltpu.VMEM((CHUNK, EMBED), jnp.float32),
            pltpu.SemaphoreType.DMA,
        ],
        compiler_params=pltpu.CompilerParams(use_tc_tiling_on_sc=False),
    )
    def gather_kernel(idx_hbm, table_hbm, out_hbm, idx_v, rows_v, sem):
        wid = lax.axis_index("s") * 2 + lax.axis_index("c")
        base = wid * rows_per_worker
        pltpu.sync_copy(idx_hbm.at[wid], idx_v)

        def body(j, carry):
            pltpu.async_copy(table_hbm.at[idx_v.at[j]], rows_v, sem).wait()
            pltpu.sync_copy(rows_v, out_hbm.at[pl.ds(base + j * CHUNK, CHUNK)])
            return carry

        lax.fori_loop(0, chunks_per_worker, body, 0)

    return gather_kernel


def kernel(x, table):
    batch, seq = x.shape
    n_rows = batch * seq
    idx = x.astype(jnp.int32).reshape(NUM_WORKERS, n_rows // (NUM_WORKERS * CHUNK), CHUNK)
    out = _make_gather(n_rows)(idx, table)
    return out.reshape(batch, seq, EMBED)


# speedup vs baseline: 1.1158x; 1.1158x over previous
import functools

import jax
import jax.numpy as jnp
from jax import lax
from jax.experimental import pallas as pl
from jax.experimental.pallas import tpu as pltpu
from jax.experimental.pallas import tpu_sc as plsc

EMBED = 64
CHUNK = 128
NBUF = 4
NUM_WORKERS = 32


@functools.lru_cache(maxsize=None)
def _make_gather(n_rows: int):
    assert n_rows % (NUM_WORKERS * CHUNK * NBUF) == 0
    chunks_per_worker = n_rows // (NUM_WORKERS * CHUNK)
    rows_per_worker = chunks_per_worker * CHUNK
    n_groups = chunks_per_worker // NBUF
    mesh = plsc.VectorSubcoreMesh(core_axis_name="c", subcore_axis_name="s")

    @functools.partial(
        pl.kernel,
        mesh=mesh,
        out_type=jax.ShapeDtypeStruct((n_rows, EMBED), jnp.float32),
        scratch_types=[
            pltpu.VMEM((chunks_per_worker, CHUNK), jnp.int32),
            pltpu.VMEM((NBUF, CHUNK, EMBED), jnp.float32),
            pltpu.SemaphoreType.DMA((NBUF,)),
        ],
        compiler_params=pltpu.CompilerParams(use_tc_tiling_on_sc=False),
    )
    def gather_kernel(idx_hbm, table_hbm, out_hbm, idx_v, rows_v, gsem):
        wid = lax.axis_index("s") * 2 + lax.axis_index("c")
        base = wid * rows_per_worker
        pltpu.sync_copy(idx_hbm.at[wid], idx_v)

        for b in range(NBUF):
            pltpu.async_copy(table_hbm.at[idx_v.at[b]], rows_v.at[b], gsem.at[b])

        def group(g, carry):
            j0 = g * NBUF
            for b in range(NBUF):
                j = j0 + b
                pltpu.make_async_copy(
                    table_hbm.at[idx_v.at[j]], rows_v.at[b], gsem.at[b]
                ).wait()
                pltpu.sync_copy(rows_v.at[b], out_hbm.at[pl.ds(base + j * CHUNK, CHUNK)])
                pltpu.async_copy(
                    table_hbm.at[idx_v.at[j + NBUF]], rows_v.at[b], gsem.at[b]
                )
            return carry

        lax.fori_loop(0, n_groups - 1, group, 0)

        j0 = (n_groups - 1) * NBUF
        for b in range(NBUF):
            j = j0 + b
            pltpu.make_async_copy(
                table_hbm.at[idx_v.at[j]], rows_v.at[b], gsem.at[b]
            ).wait()
            pltpu.sync_copy(rows_v.at[b], out_hbm.at[pl.ds(base + j * CHUNK, CHUNK)])

    return gather_kernel


def kernel(x, table):
    batch, seq = x.shape
    n_rows = batch * seq
    idx = x.astype(jnp.int32).reshape(NUM_WORKERS, n_rows // (NUM_WORKERS * CHUNK), CHUNK)
    out = _make_gather(n_rows)(idx, table)
    return out.reshape(batch, seq, EMBED)
